# baseline (speedup 1.0000x reference)
import jax
import jax.numpy as jnp
from jax import lax
from jax.experimental import pallas as pl
from jax.experimental.pallas import tpu as pltpu
from jax.experimental.pallas import tpu_sc as plsc

B = 32
HW = 56 * 56
NCH = 3
VOC = 1024
D = 128
DSEG = 32
DBLK = 4
NPAIR = DSEG // (2 * DBLK)


def _tr_body(emb_ref, out_ref):
    out_ref[...] = jnp.transpose(emb_ref[...], (0, 2, 1))


@jax.jit
def _transpose_table(embedding):
    emb3 = embedding.reshape(NCH, VOC, D)
    out = pl.pallas_call(
        _tr_body,
        out_shape=jax.ShapeDtypeStruct((NCH, D, VOC), jnp.float32),
        grid=(NCH,),
        in_specs=[pl.BlockSpec((1, VOC, D), lambda i: (i, 0, 0))],
        out_specs=pl.BlockSpec((1, D, VOC), lambda i: (i, 0, 0)),
    )(emb3)
    return out.reshape(NCH * D * VOC)


def _sc_body(in_hbm, tab_hbm, out_hbm, idxb, idxc, seg, st0, st1, sm0, sm1):
    cid = lax.axis_index("c")
    sid = lax.axis_index("s")
    b = sid * 2 + cid

    pltpu.sync_copy(in_hbm.at[pl.ds(b * (HW * NCH), HW * NCH)], idxb)

    for c in range(NCH):
        @plsc.parallel_loop(0, HW // 16, 1, unroll=2)
        def build(p16):
            base = p16 * 16
            iidx = (lax.iota(jnp.int32, 16) + base) * NCH + c
            idxc[pl.ds(base, 16)] = plsc.load_gather(idxb, [iidx])

        def dseg_body(dseg, carry):
            pltpu.sync_copy(
                tab_hbm.at[pl.ds((c * D + dseg * DSEG) * VOC, DSEG * VOC)],
                seg)

            def pair_body(pr, carry2):
                gp = (c * (D // DSEG) + dseg) * NPAIR + pr
                for ph, stage, sem in ((0, st0, sm0), (1, st1, sm1)):
                    d0 = (pr * 2 + ph) * DBLK
                    row0 = c * D + dseg * DSEG + d0
                    dst = out_hbm.at[b, pl.ds(row0, DBLK)]

                    @pl.when(gp > 0)
                    def _wait():
                        pltpu.make_async_copy(stage, dst, sem).wait()

                    @plsc.parallel_loop(0, 56, 1, unroll=2)
                    def fill(h):
                        for w0 in (0, 16, 32, 40):
                            iv = idxc[pl.ds(h * 56 + w0, 16)]
                            for dloc in range(DBLK):
                                g = plsc.load_gather(
                                    seg, [iv + (d0 + dloc) * VOC])
                                stage[dloc, h, pl.ds(w0, 16)] = g

                    pltpu.async_copy(stage, dst, sem)
                return carry2

            lax.fori_loop(0, NPAIR, pair_body, 0)
            return carry

        lax.fori_loop(0, D // DSEG, dseg_body, 0)

    for ph, stage, sem in ((0, st0, sm0), (1, st1, sm1)):
        row0 = NCH * D - 2 * DBLK + ph * DBLK
        pltpu.make_async_copy(stage, out_hbm.at[b, pl.ds(row0, DBLK)],
                              sem).wait()


@jax.jit
def _sc_call(flat_in, tab_flat):
    mesh = plsc.VectorSubcoreMesh(core_axis_name="c", subcore_axis_name="s")
    f = pl.kernel(
        _sc_body,
        out_type=jax.ShapeDtypeStruct((B, NCH * D, 56, 56), jnp.float32),
        mesh=mesh,
        scratch_types=[
            pltpu.VMEM((HW * NCH,), jnp.int32),
            pltpu.VMEM((HW,), jnp.int32),
            pltpu.VMEM((DSEG * VOC,), jnp.float32),
            pltpu.VMEM((DBLK, 56, 56), jnp.float32),
            pltpu.VMEM((DBLK, 56, 56), jnp.float32),
            pltpu.SemaphoreType.DMA,
            pltpu.SemaphoreType.DMA,
        ],
        compiler_params=pltpu.CompilerParams(needs_layout_passes=False),
    )
    return f(flat_in, tab_flat)


def kernel(inputs, embedding):
    tab_flat = _transpose_table(embedding)
    flat_in = inputs.reshape(B * HW * NCH)
    return _sc_call(flat_in, tab_flat)


# speedup vs baseline: 3.3998x; 3.3998x over previous
import jax
import jax.numpy as jnp
from jax import lax
from jax.experimental import pallas as pl
from jax.experimental.pallas import tpu as pltpu
from jax.experimental.pallas import tpu_sc as plsc

B = 32
H = 56
W = 56
NCH = 3
VOC = 1024
D = 128
RCHUNK = 2
NCHUNK = H // RCHUNK


def _sc_body(in_hbm, emb_hbm, out_hbm, pl0, pl1, pl2, *rest):
    oixs = rest[:12]
    st0, st1, gsm0, gsm1, osm0, osm1 = rest[12:]
    cid = lax.axis_index("c")
    sid = lax.axis_index("s")
    b = sid * 2 + cid
    planes = (pl0, pl1, pl2)

    for c in range(NCH):
        pltpu.sync_copy(in_hbm.at[b, c], planes[c])

    def pair_body(it, carry):
        for ph, stage, gsm, osm in ((0, st0, gsm0, osm0),
                                    (1, st1, gsm1, osm1)):
            chunk = it * 2 + ph
            h0 = chunk * RCHUNK
            dst = out_hbm.at[b, pl.ds(h0, RCHUNK)]

            @pl.when(it > 0)
            def _wait_out():
                pltpu.make_async_copy(stage, dst, osm).wait()

            for hh in range(RCHUNK):
                for c in range(NCH):
                    oix = oixs[(ph * RCHUNK + hh) * NCH + c]
                    for w0 in (0, 16, 32, 40):
                        iv = planes[c][h0 + hh, pl.ds(w0, 16)]
                        oix[pl.ds(w0, 16)] = iv + c * VOC

            for hh in range(RCHUNK):
                for c in range(NCH):
                    oix = oixs[(ph * RCHUNK + hh) * NCH + c]
                    pltpu.async_copy(
                        emb_hbm.at[oix],
                        stage.at[hh, :, pl.ds(c * D, D)],
                        gsm)
            for hh in range(RCHUNK):
                for c in range(NCH):
                    oix = oixs[(ph * RCHUNK + hh) * NCH + c]
                    pltpu.make_async_copy(
                        emb_hbm.at[oix],
                        stage.at[hh, :, pl.ds(c * D, D)],
                        gsm).wait()

            pltpu.async_copy(stage, dst, osm)
        return carry

    lax.fori_loop(0, NCHUNK // 2, pair_body, 0)

    for ph, stage, osm in ((0, st0, osm0), (1, st1, osm1)):
        h0 = H - 2 * RCHUNK + ph * RCHUNK
        pltpu.make_async_copy(stage, out_hbm.at[b, pl.ds(h0, RCHUNK)],
                              osm).wait()


@jax.jit
def _sc_call(in_t, embedding):
    mesh = plsc.VectorSubcoreMesh(core_axis_name="c", subcore_axis_name="s")
    f = pl.kernel(
        _sc_body,
        out_type=jax.ShapeDtypeStruct((B, H, W, NCH * D), jnp.float32),
        mesh=mesh,
        scratch_types=[
            pltpu.VMEM((H, W), jnp.int32),
            pltpu.VMEM((H, W), jnp.int32),
            pltpu.VMEM((H, W), jnp.int32),
        ] + [pltpu.VMEM((W,), jnp.int32)] * 12 + [
            pltpu.VMEM((RCHUNK, W, NCH * D), jnp.float32),
            pltpu.VMEM((RCHUNK, W, NCH * D), jnp.float32),
            pltpu.SemaphoreType.DMA,
            pltpu.SemaphoreType.DMA,
            pltpu.SemaphoreType.DMA,
            pltpu.SemaphoreType.DMA,
        ],
        compiler_params=pltpu.CompilerParams(needs_layout_passes=False),
    )
    return f(in_t, embedding)


def kernel(inputs, embedding):
    in_t = jnp.transpose(inputs, (0, 3, 1, 2))
    out = _sc_call(in_t, embedding)
    return jnp.transpose(out, (0, 3, 1, 2))


# speedup vs baseline: 3.4020x; 1.0007x over previous
import jax
import jax.numpy as jnp
from jax import lax
from jax.experimental import pallas as pl
from jax.experimental.pallas import tpu as pltpu
from jax.experimental.pallas import tpu_sc as plsc

B = 32
H = 56
W = 56
NCH = 3
VOC = 1024
D = 128
RCH = 2
PCH = RCH * W
NCHUNK = H // RCH


def _sc_body(in_hbm, emb_hbm, out_hbm, pl0, pl1, pl2, *rest):
    oixs = rest[:6]
    st0, st1, gsm0, gsm1, osm0, osm1 = rest[6:]
    cid = lax.axis_index("c")
    sid = lax.axis_index("s")
    b = sid * 2 + cid
    planes = (pl0, pl1, pl2)

    for c in range(NCH):
        pltpu.sync_copy(in_hbm.at[b, c], planes[c])

    def prep_and_gather(it, ph, stage, gsm):
        h0 = (it * 2 + ph) * RCH
        for c in range(NCH):
            oix = oixs[ph * NCH + c]
            for r in range(RCH):
                for w0 in (0, 16, 32, 40):
                    iv = planes[c][h0 + r, pl.ds(w0, 16)]
                    oix[pl.ds(r * W + w0, 16)] = iv + c * VOC
        for c in range(NCH):
            pltpu.async_copy(
                emb_hbm.at[oixs[ph * NCH + c]],
                stage.at[:, pl.ds(c * D, D)],
                gsm)

    def wait_gathers(ph, stage, gsm):
        for c in range(NCH):
            pltpu.make_async_copy(
                emb_hbm.at[oixs[ph * NCH + c]],
                stage.at[:, pl.ds(c * D, D)],
                gsm).wait()

    def out_dst(chunk):
        return out_hbm.at[b, pl.ds(chunk * PCH, PCH)]

    def pair_body(it, carry):
        @pl.when(it > 0)
        def _drain_prev_odd():
            wait_gathers(1, st1, gsm1)
            pltpu.async_copy(st1, out_dst(it * 2 - 1), osm1)
            pltpu.make_async_copy(st0, out_dst(0), osm0).wait()

        prep_and_gather(it, 0, st0, gsm0)

        wait_gathers(0, st0, gsm0)
        pltpu.async_copy(st0, out_dst(it * 2), osm0)

        @pl.when(it > 0)
        def _wait_old_odd_out():
            pltpu.make_async_copy(st1, out_dst(0), osm1).wait()

        prep_and_gather(it, 1, st1, gsm1)
        return carry

    lax.fori_loop(0, NCHUNK // 2, pair_body, 0)

    wait_gathers(1, st1, gsm1)
    pltpu.async_copy(st1, out_dst(NCHUNK - 1), osm1)
    pltpu.make_async_copy(st0, out_dst(0), osm0).wait()
    pltpu.make_async_copy(st1, out_dst(0), osm1).wait()


@jax.jit
def _sc_call(in_t, embedding):
    mesh = plsc.VectorSubcoreMesh(core_axis_name="c", subcore_axis_name="s")
    f = pl.kernel(
        _sc_body,
        out_type=jax.ShapeDtypeStruct((B, H * W, NCH * D), jnp.float32),
        mesh=mesh,
        scratch_types=[
            pltpu.VMEM((H, W), jnp.int32),
            pltpu.VMEM((H, W), jnp.int32),
            pltpu.VMEM((H, W), jnp.int32),
        ] + [pltpu.VMEM((PCH,), jnp.int32)] * 6 + [
            pltpu.VMEM((PCH, NCH * D), jnp.float32),
            pltpu.VMEM((PCH, NCH * D), jnp.float32),
            pltpu.SemaphoreType.DMA,
            pltpu.SemaphoreType.DMA,
            pltpu.SemaphoreType.DMA,
            pltpu.SemaphoreType.DMA,
        ],
        compiler_params=pltpu.CompilerParams(needs_layout_passes=False),
    )
    return f(in_t, embedding)


def kernel(inputs, embedding):
    in_t = jnp.transpose(inputs, (0, 3, 1, 2))
    out = _sc_call(in_t, embedding)
    out4 = out.reshape(B, H, W, NCH * D)
    return jnp.transpose(out4, (0, 3, 1, 2))
